# Initial kernel scaffold; baseline (speedup 1.0000x reference)
import jax
import jax.numpy as jnp
from jax.experimental import pallas as pl


def kernel(idxs, embedding_weight):
    raise NotImplementedError("write your pallas kernel here")


# speedup vs baseline: 6.2520x; 6.2520x over previous
import functools

import jax
import jax.numpy as jnp
from jax import lax
from jax.experimental import pallas as pl
from jax.experimental.pallas import tpu as pltpu
from jax.experimental.pallas import tpu_sc as plsc

LANE = 128
D = 16
K = 16
NUM_WORKERS = 32


def _emb_body(idx_hbm, table_hbm, out_hbm, idx_v, rows_v, sem):
    n_rows = idx_hbm.shape[0]
    nc = 2
    wid = lax.axis_index("s") * nc + lax.axis_index("c")
    rows_per_tile = n_rows // NUM_WORKERS
    n_chunks = rows_per_tile // K
    base = wid * rows_per_tile

    def chunk(g, carry):
        r0 = base + g * K
        pltpu.sync_copy(idx_hbm.at[pl.ds(r0, K)], idx_v)
        copies = [
            pltpu.async_copy(table_hbm.at[idx_v.at[j]], rows_v.at[j], sem)
            for j in range(K)
        ]
        for c in copies:
            c.wait()
        pltpu.sync_copy(rows_v, out_hbm.at[pl.ds(r0, K)])
        return carry

    lax.fori_loop(0, n_chunks, chunk, 0)


def kernel(idxs, embedding_weight):
    b0, b1 = idxs.shape
    n_rows = (b0 * b1) // LANE
    idx2d = idxs.reshape(n_rows, LANE)

    call = functools.partial(
        pl.kernel,
        mesh=plsc.VectorSubcoreMesh(core_axis_name="c", subcore_axis_name="s"),
        out_type=jax.ShapeDtypeStruct((n_rows, LANE, D), jnp.float32),
        scratch_types=[
            pltpu.VMEM((K, LANE), jnp.int32),
            pltpu.VMEM((K, LANE, D), jnp.float32),
            pltpu.SemaphoreType.DMA,
        ],
        compiler_params=pltpu.CompilerParams(use_tc_tiling_on_sc=False),
    )(_emb_body)

    out = call(idx2d, embedding_weight)
    return out.reshape(b0, b1, D)
